# Initial kernel scaffold; baseline (speedup 1.0000x reference)
import jax
import jax.numpy as jnp
from jax.experimental import pallas as pl


def kernel(Y, X, R_pred, t_pred, R_gt, t_gt, X_hat, delta):
    raise NotImplementedError("write your pallas kernel here")


# speedup vs baseline: 26.1224x; 26.1224x over previous
import jax
import jax.numpy as jnp
from jax import lax
from jax.experimental import pallas as pl
from jax.experimental.pallas import tpu as pltpu

_B, _N, _M, _K = 2, 2048, 1024, 5
_TILE = 256
_T = _N // _TILE
_INF = 3.0e38


def _body(Yr_ref, Ya_ref, X_ref, Rp_ref, tp_ref, Rg_ref, tg_ref,
          Xhr_ref, Xha_ref, dlr_ref, dla_ref, out_ref):
    t = pl.program_id(1)
    f32 = jnp.float32
    i32 = jnp.int32

    yrow = Yr_ref[0]
    yall = Ya_ref[0]
    x = X_ref[0]
    Rp = Rp_ref[0]
    tp = tp_ref[0]
    Rg = Rg_ref[0]
    tg = tg_ref[0]
    xh_r = Xhr_ref[0]
    xh_a = Xha_ref[0]
    de_r = dlr_ref[0]
    de_a = dla_ref[0]

    dn = (((1,), (1,)), ((), ()))

    yrig_r = lax.dot_general(yrow, Rp, dn) + tp
    yrig_a = lax.dot_general(yall, Rp, dn) + tp

    nr = jnp.sum(yrig_r * yrig_r, axis=1, keepdims=True)
    na = jnp.sum(yrig_a * yrig_a, axis=1, keepdims=True)
    ones_r = jnp.ones((_TILE, 1), f32)
    ones_a = jnp.ones((_N, 1), f32)
    U = jnp.concatenate([-2.0 * yrig_r, nr, ones_r], axis=1)
    V = jnp.concatenate([yrig_a, ones_a, na], axis=1)
    d = lax.dot_general(U, V, dn)

    row_id = t * _TILE + lax.broadcasted_iota(i32, (_TILE, _N), 0)
    col_id = lax.broadcasted_iota(i32, (_TILE, _N), 1)
    d = jnp.where(row_id == col_id, _INF, d)

    selmask = jnp.zeros((_TILE, _N), f32)
    for _ in range(_K):
        m = jnp.min(d, axis=1, keepdims=True)
        cand = jnp.where(d == m, col_id, _N)
        am = jnp.min(cand, axis=1, keepdims=True)
        oh = col_id == am
        selmask = jnp.where(oh, 1.0, selmask)
        d = jnp.where(oh, _INF, d)

    D_a = xh_a - yrig_a
    D2_a = jnp.sum(D_a * D_a, axis=1, keepdims=True)
    F = jnp.concatenate([D_a, D2_a, de_a], axis=1)
    sel = lax.dot_general(selmask, F, (((1,), (0,)), ((), ())))
    S1 = sel[:, 0:3]
    S2 = sel[:, 3:4]
    Sd = sel[:, 4:7]

    D_r = xh_r - yrig_r
    deform_s = (jnp.sum(S2) - 2.0 * jnp.sum(D_r * S1)
                + f32(_K) * jnp.sum(D_r * D_r))
    lap = de_r - Sd * f32(1.0 / _K)
    lap_s = jnp.sum(lap * lap)
    disp_s = jnp.sum(de_r * de_r)

    nxh = jnp.sum(xh_r * xh_r, axis=1, keepdims=True)
    nx = jnp.sum(x * x, axis=1, keepdims=True)
    ones_m = jnp.ones((_M, 1), f32)
    UA = jnp.concatenate([-2.0 * xh_r, nxh, ones_r], axis=1)
    VA = jnp.concatenate([x, ones_m, nx], axis=1)
    dA = lax.dot_general(UA, VA, dn)
    colA = lax.broadcasted_iota(i32, (_TILE, _M), 1)
    align_s = f32(0.0)
    for _ in range(_K):
        mA = jnp.min(dA, axis=1, keepdims=True)
        align_s = align_s + jnp.sum(mA)
        candA = jnp.where(dA == mA, colA, _M)
        amA = jnp.min(candA, axis=1, keepdims=True)
        dA = jnp.where(colA == amA, _INF, dA)

    E = lax.dot_general(yrow, Rp - Rg, dn) + (tp - tg)
    rmse_s = jnp.sum(E * E)

    Rd = lax.dot_general(Rp, Rg, (((0,), (0,)), ((), ())))
    eye = (lax.broadcasted_iota(i32, (3, 3), 0)
           == lax.broadcasted_iota(i32, (3, 3), 1))
    tr = jnp.sum(jnp.where(eye, Rd, 0.0))
    dtr = tp - tg
    trans_sq = jnp.sum(dtr * dtr)
    gate = jnp.where(t == 0, f32(1.0), f32(0.0))

    lane = lax.broadcasted_iota(i32, (1, 1, 128), 2)
    vals = (jnp.where(lane == 0, align_s, 0.0)
            + jnp.where(lane == 1, deform_s, 0.0)
            + jnp.where(lane == 2, lap_s, 0.0)
            + jnp.where(lane == 3, disp_s, 0.0)
            + jnp.where(lane == 4, rmse_s, 0.0)
            + jnp.where(lane == 5, gate * tr, 0.0)
            + jnp.where(lane == 6, gate * trans_sq, 0.0))

    @pl.when(t == 0)
    def _init():
        out_ref[...] = jnp.zeros_like(out_ref)

    out_ref[...] += vals


def kernel(Y, X, R_pred, t_pred, R_gt, t_gt, X_hat, delta):
    f32 = jnp.float32
    tp3 = t_pred.reshape(_B, 1, 3).astype(f32)
    tg3 = t_gt.reshape(_B, 1, 3).astype(f32)

    rows = lambda b, t: (b, t, 0)
    full = lambda b, t: (b, 0, 0)

    out = pl.pallas_call(
        _body,
        grid=(_B, _T),
        in_specs=[
            pl.BlockSpec((1, _TILE, 3), rows),
            pl.BlockSpec((1, _N, 3), full),
            pl.BlockSpec((1, _M, 3), full),
            pl.BlockSpec((1, 3, 3), full),
            pl.BlockSpec((1, 1, 3), full),
            pl.BlockSpec((1, 3, 3), full),
            pl.BlockSpec((1, 1, 3), full),
            pl.BlockSpec((1, _TILE, 3), rows),
            pl.BlockSpec((1, _N, 3), full),
            pl.BlockSpec((1, _TILE, 3), rows),
            pl.BlockSpec((1, _N, 3), full),
        ],
        out_specs=pl.BlockSpec((1, 1, 128), full),
        out_shape=jax.ShapeDtypeStruct((_B, 1, 128), f32),
    )(Y, Y, X, R_pred, tp3, R_gt, tg3, X_hat, X_hat, delta, delta)

    o = out[:, 0, :]
    NK = f32(_N * _K)
    L_align = o[:, 0] / NK
    L_deform = o[:, 1] / NK
    L_lap = o[:, 2] / f32(_N)
    L_disp = o[:, 3] / f32(_N)
    L_rmse = jnp.sqrt(o[:, 4] / f32(_N))
    tr = o[:, 5]
    trans_sq = o[:, 6]
    c = jnp.clip((tr - 1.0) / 2.0, -1.0 + 1e-07, 1.0 - 1e-07)
    L_rot = jnp.arccos(c)
    L_trans = jnp.sqrt(trans_sq)
    total = (L_rot + L_trans + L_rmse + L_align
             + 0.01 * L_disp + 0.1 * L_deform + 0.1 * L_lap)
    return total.mean()


# speedup vs baseline: 39.4343x; 1.5096x over previous
import jax
import jax.numpy as jnp
from jax import lax
from jax.experimental import pallas as pl
from jax.experimental.pallas import tpu as pltpu

_B, _N, _M, _K = 2, 2048, 1024, 5
_TILE = 512
_T = _N // _TILE
_INF = 3.0e38


def _body(Yr_ref, Ya_ref, X_ref, Rp_ref, tp_ref, Rg_ref, tg_ref,
          Xhr_ref, Xha_ref, dlr_ref, dla_ref, out_ref):
    t = pl.program_id(1)
    f32 = jnp.float32
    i32 = jnp.int32

    yrow = Yr_ref[0]
    yall = Ya_ref[0]
    x = X_ref[0]
    Rp = Rp_ref[0]
    tp = tp_ref[0]
    Rg = Rg_ref[0]
    tg = tg_ref[0]
    xh_r = Xhr_ref[0]
    xh_a = Xha_ref[0]
    de_r = dlr_ref[0]
    de_a = dla_ref[0]

    dn = (((1,), (1,)), ((), ()))

    yrig_r = lax.dot_general(yrow, Rp, dn) + tp
    yrig_a = lax.dot_general(yall, Rp, dn) + tp

    nr = jnp.sum(yrig_r * yrig_r, axis=1, keepdims=True)
    na = jnp.sum(yrig_a * yrig_a, axis=1, keepdims=True)
    ones_r = jnp.ones((_TILE, 1), f32)
    ones_a = jnp.ones((_N, 1), f32)
    U = jnp.concatenate([-2.0 * yrig_r, nr, ones_r], axis=1)
    V = jnp.concatenate([yrig_a, ones_a, na], axis=1)
    d = lax.dot_general(U, V, dn)

    row_id = t * _TILE + lax.broadcasted_iota(i32, (_TILE, _N), 0)
    col_id = lax.broadcasted_iota(i32, (_TILE, _N), 1)
    d = jnp.where(row_id == col_id, _INF, d)

    selmask = jnp.zeros((_TILE, _N), f32)
    for _ in range(_K):
        m = jnp.min(d, axis=1, keepdims=True)
        eq = d == m
        selmask = jnp.where(eq, 1.0, selmask)
        d = jnp.where(eq, _INF, d)

    D_a = xh_a - yrig_a
    D2_a = jnp.sum(D_a * D_a, axis=1, keepdims=True)
    F = jnp.concatenate([D_a, D2_a, de_a], axis=1)
    sel = lax.dot_general(selmask, F, (((1,), (0,)), ((), ())))
    S1 = sel[:, 0:3]
    S2 = sel[:, 3:4]
    Sd = sel[:, 4:7]

    D_r = xh_r - yrig_r
    deform_s = (jnp.sum(S2) - 2.0 * jnp.sum(D_r * S1)
                + f32(_K) * jnp.sum(D_r * D_r))
    lap = de_r - Sd * f32(1.0 / _K)
    lap_s = jnp.sum(lap * lap)
    disp_s = jnp.sum(de_r * de_r)

    nxh = jnp.sum(xh_r * xh_r, axis=1, keepdims=True)
    nx = jnp.sum(x * x, axis=1, keepdims=True)
    ones_m = jnp.ones((_M, 1), f32)
    UA = jnp.concatenate([-2.0 * xh_r, nxh, ones_r], axis=1)
    VA = jnp.concatenate([x, ones_m, nx], axis=1)
    dA = lax.dot_general(UA, VA, dn)
    align_s = f32(0.0)
    for _ in range(_K):
        mA = jnp.min(dA, axis=1, keepdims=True)
        align_s = align_s + jnp.sum(mA)
        dA = jnp.where(dA == mA, _INF, dA)

    E = lax.dot_general(yrow, Rp - Rg, dn) + (tp - tg)
    rmse_s = jnp.sum(E * E)

    Rd = lax.dot_general(Rp, Rg, (((0,), (0,)), ((), ())))
    eye = (lax.broadcasted_iota(i32, (3, 3), 0)
           == lax.broadcasted_iota(i32, (3, 3), 1))
    tr = jnp.sum(jnp.where(eye, Rd, 0.0))
    dtr = tp - tg
    trans_sq = jnp.sum(dtr * dtr)
    gate = jnp.where(t == 0, f32(1.0), f32(0.0))

    lane = lax.broadcasted_iota(i32, (1, 1, 128), 2)
    vals = (jnp.where(lane == 0, align_s, 0.0)
            + jnp.where(lane == 1, deform_s, 0.0)
            + jnp.where(lane == 2, lap_s, 0.0)
            + jnp.where(lane == 3, disp_s, 0.0)
            + jnp.where(lane == 4, rmse_s, 0.0)
            + jnp.where(lane == 5, gate * tr, 0.0)
            + jnp.where(lane == 6, gate * trans_sq, 0.0))

    @pl.when(t == 0)
    def _init():
        out_ref[...] = jnp.zeros_like(out_ref)

    out_ref[...] += vals


def kernel(Y, X, R_pred, t_pred, R_gt, t_gt, X_hat, delta):
    f32 = jnp.float32
    tp3 = t_pred.reshape(_B, 1, 3).astype(f32)
    tg3 = t_gt.reshape(_B, 1, 3).astype(f32)

    rows = lambda b, t: (b, t, 0)
    full = lambda b, t: (b, 0, 0)

    out = pl.pallas_call(
        _body,
        grid=(_B, _T),
        in_specs=[
            pl.BlockSpec((1, _TILE, 3), rows),
            pl.BlockSpec((1, _N, 3), full),
            pl.BlockSpec((1, _M, 3), full),
            pl.BlockSpec((1, 3, 3), full),
            pl.BlockSpec((1, 1, 3), full),
            pl.BlockSpec((1, 3, 3), full),
            pl.BlockSpec((1, 1, 3), full),
            pl.BlockSpec((1, _TILE, 3), rows),
            pl.BlockSpec((1, _N, 3), full),
            pl.BlockSpec((1, _TILE, 3), rows),
            pl.BlockSpec((1, _N, 3), full),
        ],
        out_specs=pl.BlockSpec((1, 1, 128), full),
        out_shape=jax.ShapeDtypeStruct((_B, 1, 128), f32),
    )(Y, Y, X, R_pred, tp3, R_gt, tg3, X_hat, X_hat, delta, delta)

    o = out[:, 0, :]
    NK = f32(_N * _K)
    L_align = o[:, 0] / NK
    L_deform = o[:, 1] / NK
    L_lap = o[:, 2] / f32(_N)
    L_disp = o[:, 3] / f32(_N)
    L_rmse = jnp.sqrt(o[:, 4] / f32(_N))
    tr = o[:, 5]
    trans_sq = o[:, 6]
    c = jnp.clip((tr - 1.0) / 2.0, -1.0 + 1e-07, 1.0 - 1e-07)
    L_rot = jnp.arccos(c)
    L_trans = jnp.sqrt(trans_sq)
    total = (L_rot + L_trans + L_rmse + L_align
             + 0.01 * L_disp + 0.1 * L_deform + 0.1 * L_lap)
    return total.mean()


# speedup vs baseline: 47.5926x; 1.2069x over previous
import jax
import jax.numpy as jnp
from jax import lax
from jax.experimental import pallas as pl
from jax.experimental.pallas import tpu as pltpu

_B, _N, _M, _K = 2, 2048, 1024, 5
_TILE = 1024
_T = _N // _TILE
_INF = 3.0e38
_SELF = 1.0e38
_MID = 2.0e38


def _body(Yr_ref, Ya_ref, X_ref, Rp_ref, tp_ref, Rg_ref, tg_ref,
          Xhr_ref, Xha_ref, dlr_ref, dla_ref, out_ref):
    t = pl.program_id(1)
    f32 = jnp.float32
    i32 = jnp.int32

    yrow = Yr_ref[0]
    yall = Ya_ref[0]
    x = X_ref[0]
    Rp = Rp_ref[0]
    tp = tp_ref[0]
    Rg = Rg_ref[0]
    tg = tg_ref[0]
    xh_r = Xhr_ref[0]
    xh_a = Xha_ref[0]
    de_r = dlr_ref[0]
    de_a = dla_ref[0]

    dn = (((1,), (1,)), ((), ()))

    yrig_r = lax.dot_general(yrow, Rp, dn) + tp
    yrig_a = lax.dot_general(yall, Rp, dn) + tp

    nr = jnp.sum(yrig_r * yrig_r, axis=1, keepdims=True)
    na = jnp.sum(yrig_a * yrig_a, axis=1, keepdims=True)
    ones_r = jnp.ones((_TILE, 1), f32)
    ones_a = jnp.ones((_N, 1), f32)
    U = jnp.concatenate([-2.0 * yrig_r, nr, ones_r], axis=1)
    V = jnp.concatenate([yrig_a, ones_a, na], axis=1)
    d = lax.dot_general(U, V, dn)

    row_id = t * _TILE + lax.broadcasted_iota(i32, (_TILE, _N), 0)
    col_id = lax.broadcasted_iota(i32, (_TILE, _N), 1)
    d = jnp.where(row_id == col_id, _SELF, d)

    for _ in range(_K):
        m = jnp.min(d, axis=1, keepdims=True)
        d = jnp.where(d == m, _INF, d)
    selmask = jnp.where(d > _MID, 1.0, 0.0).astype(f32)

    D_a = xh_a - yrig_a
    D2_a = jnp.sum(D_a * D_a, axis=1, keepdims=True)
    F = jnp.concatenate([D_a, D2_a, de_a], axis=1)
    sel = lax.dot_general(selmask, F, (((1,), (0,)), ((), ())))
    S1 = sel[:, 0:3]
    S2 = sel[:, 3:4]
    Sd = sel[:, 4:7]

    D_r = xh_r - yrig_r
    deform_s = (jnp.sum(S2) - 2.0 * jnp.sum(D_r * S1)
                + f32(_K) * jnp.sum(D_r * D_r))
    lap = de_r - Sd * f32(1.0 / _K)
    lap_s = jnp.sum(lap * lap)
    disp_s = jnp.sum(de_r * de_r)

    nxh = jnp.sum(xh_r * xh_r, axis=1, keepdims=True)
    nx = jnp.sum(x * x, axis=1, keepdims=True)
    ones_m = jnp.ones((_M, 1), f32)
    UA = jnp.concatenate([-2.0 * xh_r, nxh, ones_r], axis=1)
    VA = jnp.concatenate([x, ones_m, nx], axis=1)
    dA = lax.dot_general(UA, VA, dn)
    align_s = f32(0.0)
    for _ in range(_K):
        mA = jnp.min(dA, axis=1, keepdims=True)
        align_s = align_s + jnp.sum(mA)
        dA = jnp.where(dA == mA, _INF, dA)

    E = lax.dot_general(yrow, Rp - Rg, dn) + (tp - tg)
    rmse_s = jnp.sum(E * E)

    Rd = lax.dot_general(Rp, Rg, (((0,), (0,)), ((), ())))
    eye = (lax.broadcasted_iota(i32, (3, 3), 0)
           == lax.broadcasted_iota(i32, (3, 3), 1))
    tr = jnp.sum(jnp.where(eye, Rd, 0.0))
    dtr = tp - tg
    trans_sq = jnp.sum(dtr * dtr)
    gate = jnp.where(t == 0, f32(1.0), f32(0.0))

    lane = lax.broadcasted_iota(i32, (1, 1, 128), 2)
    vals = (jnp.where(lane == 0, align_s, 0.0)
            + jnp.where(lane == 1, deform_s, 0.0)
            + jnp.where(lane == 2, lap_s, 0.0)
            + jnp.where(lane == 3, disp_s, 0.0)
            + jnp.where(lane == 4, rmse_s, 0.0)
            + jnp.where(lane == 5, gate * tr, 0.0)
            + jnp.where(lane == 6, gate * trans_sq, 0.0))

    @pl.when(t == 0)
    def _init():
        out_ref[...] = jnp.zeros_like(out_ref)

    out_ref[...] += vals


def kernel(Y, X, R_pred, t_pred, R_gt, t_gt, X_hat, delta):
    f32 = jnp.float32
    tp3 = t_pred.reshape(_B, 1, 3).astype(f32)
    tg3 = t_gt.reshape(_B, 1, 3).astype(f32)

    rows = lambda b, t: (b, t, 0)
    full = lambda b, t: (b, 0, 0)

    out = pl.pallas_call(
        _body,
        grid=(_B, _T),
        in_specs=[
            pl.BlockSpec((1, _TILE, 3), rows),
            pl.BlockSpec((1, _N, 3), full),
            pl.BlockSpec((1, _M, 3), full),
            pl.BlockSpec((1, 3, 3), full),
            pl.BlockSpec((1, 1, 3), full),
            pl.BlockSpec((1, 3, 3), full),
            pl.BlockSpec((1, 1, 3), full),
            pl.BlockSpec((1, _TILE, 3), rows),
            pl.BlockSpec((1, _N, 3), full),
            pl.BlockSpec((1, _TILE, 3), rows),
            pl.BlockSpec((1, _N, 3), full),
        ],
        out_specs=pl.BlockSpec((1, 1, 128), full),
        out_shape=jax.ShapeDtypeStruct((_B, 1, 128), f32),
    )(Y, Y, X, R_pred, tp3, R_gt, tg3, X_hat, X_hat, delta, delta)

    o = out[:, 0, :]
    NK = f32(_N * _K)
    L_align = o[:, 0] / NK
    L_deform = o[:, 1] / NK
    L_lap = o[:, 2] / f32(_N)
    L_disp = o[:, 3] / f32(_N)
    L_rmse = jnp.sqrt(o[:, 4] / f32(_N))
    tr = o[:, 5]
    trans_sq = o[:, 6]
    c = jnp.clip((tr - 1.0) / 2.0, -1.0 + 1e-07, 1.0 - 1e-07)
    L_rot = jnp.arccos(c)
    L_trans = jnp.sqrt(trans_sq)
    total = (L_rot + L_trans + L_rmse + L_align
             + 0.01 * L_disp + 0.1 * L_deform + 0.1 * L_lap)
    return total.mean()


# speedup vs baseline: 48.8904x; 1.0273x over previous
import jax
import jax.numpy as jnp
from jax import lax
from jax.experimental import pallas as pl
from jax.experimental.pallas import tpu as pltpu

_B, _N, _M, _K = 2, 2048, 1024, 5
_TILE = 1024
_T = _N // _TILE
_INF = 3.0e38
_SELF = 1.0e38
_MID = 2.0e38


def _body(Yr_ref, Ya_ref, X_ref, Rp_ref, tp_ref, Rg_ref, tg_ref,
          Xhr_ref, Xha_ref, dlr_ref, dla_ref, out_ref):
    t = pl.program_id(1)
    f32 = jnp.float32
    i32 = jnp.int32

    yrow = Yr_ref[0]
    yall = Ya_ref[0]
    x = X_ref[0]
    Rp = Rp_ref[0]
    tp = tp_ref[0]
    Rg = Rg_ref[0]
    tg = tg_ref[0]
    xh_r = Xhr_ref[0]
    xh_a = Xha_ref[0]
    de_r = dlr_ref[0]
    de_a = dla_ref[0]

    dn = (((1,), (1,)), ((), ()))

    yrig_r = lax.dot_general(yrow, Rp, dn) + tp
    yrig_a = lax.dot_general(yall, Rp, dn) + tp

    nr = jnp.sum(yrig_r * yrig_r, axis=1, keepdims=True)
    na = jnp.sum(yrig_a * yrig_a, axis=1, keepdims=True)
    ones_r = jnp.ones((_TILE, 1), f32)
    ones_a = jnp.ones((_N, 1), f32)
    U = jnp.concatenate([-2.0 * yrig_r, nr, ones_r], axis=1)
    V = jnp.concatenate([yrig_a, ones_a, na], axis=1)
    d = lax.dot_general(U, V, dn)

    row_id = t * _TILE + lax.broadcasted_iota(i32, (_TILE, _N), 0)
    col_id = lax.broadcasted_iota(i32, (_TILE, _N), 1)
    d = jnp.where(row_id == col_id, _SELF, d)

    m = jnp.min(d, axis=1, keepdims=True)
    for _ in range(_K - 1):
        m = jnp.min(jnp.where(d <= m, _INF, d), axis=1, keepdims=True)
    selmask = jnp.where(d <= m, 1.0, 0.0).astype(f32)

    D_a = xh_a - yrig_a
    D2_a = jnp.sum(D_a * D_a, axis=1, keepdims=True)
    F = jnp.concatenate([D_a, D2_a, de_a], axis=1)
    sel = lax.dot_general(selmask, F, (((1,), (0,)), ((), ())))
    S1 = sel[:, 0:3]
    S2 = sel[:, 3:4]
    Sd = sel[:, 4:7]

    D_r = xh_r - yrig_r
    deform_s = (jnp.sum(S2) - 2.0 * jnp.sum(D_r * S1)
                + f32(_K) * jnp.sum(D_r * D_r))
    lap = de_r - Sd * f32(1.0 / _K)
    lap_s = jnp.sum(lap * lap)
    disp_s = jnp.sum(de_r * de_r)

    nxh = jnp.sum(xh_r * xh_r, axis=1, keepdims=True)
    nx = jnp.sum(x * x, axis=1, keepdims=True)
    ones_m = jnp.ones((_M, 1), f32)
    UA = jnp.concatenate([-2.0 * xh_r, nxh, ones_r], axis=1)
    VA = jnp.concatenate([x, ones_m, nx], axis=1)
    dA = lax.dot_general(UA, VA, dn)
    mA = jnp.min(dA, axis=1, keepdims=True)
    align_s = jnp.sum(mA)
    for _ in range(_K - 1):
        mA = jnp.min(jnp.where(dA <= mA, _INF, dA), axis=1, keepdims=True)
        align_s = align_s + jnp.sum(mA)

    E = lax.dot_general(yrow, Rp - Rg, dn) + (tp - tg)
    rmse_s = jnp.sum(E * E)

    Rd = lax.dot_general(Rp, Rg, (((0,), (0,)), ((), ())))
    eye = (lax.broadcasted_iota(i32, (3, 3), 0)
           == lax.broadcasted_iota(i32, (3, 3), 1))
    tr = jnp.sum(jnp.where(eye, Rd, 0.0))
    dtr = tp - tg
    trans_sq = jnp.sum(dtr * dtr)
    gate = jnp.where(t == 0, f32(1.0), f32(0.0))

    lane = lax.broadcasted_iota(i32, (1, 1, 128), 2)
    vals = (jnp.where(lane == 0, align_s, 0.0)
            + jnp.where(lane == 1, deform_s, 0.0)
            + jnp.where(lane == 2, lap_s, 0.0)
            + jnp.where(lane == 3, disp_s, 0.0)
            + jnp.where(lane == 4, rmse_s, 0.0)
            + jnp.where(lane == 5, gate * tr, 0.0)
            + jnp.where(lane == 6, gate * trans_sq, 0.0))

    @pl.when(t == 0)
    def _init():
        out_ref[...] = jnp.zeros_like(out_ref)

    out_ref[...] += vals


def kernel(Y, X, R_pred, t_pred, R_gt, t_gt, X_hat, delta):
    f32 = jnp.float32
    tp3 = t_pred.reshape(_B, 1, 3).astype(f32)
    tg3 = t_gt.reshape(_B, 1, 3).astype(f32)

    rows = lambda b, t: (b, t, 0)
    full = lambda b, t: (b, 0, 0)

    out = pl.pallas_call(
        _body,
        grid=(_B, _T),
        in_specs=[
            pl.BlockSpec((1, _TILE, 3), rows),
            pl.BlockSpec((1, _N, 3), full),
            pl.BlockSpec((1, _M, 3), full),
            pl.BlockSpec((1, 3, 3), full),
            pl.BlockSpec((1, 1, 3), full),
            pl.BlockSpec((1, 3, 3), full),
            pl.BlockSpec((1, 1, 3), full),
            pl.BlockSpec((1, _TILE, 3), rows),
            pl.BlockSpec((1, _N, 3), full),
            pl.BlockSpec((1, _TILE, 3), rows),
            pl.BlockSpec((1, _N, 3), full),
        ],
        out_specs=pl.BlockSpec((1, 1, 128), full),
        out_shape=jax.ShapeDtypeStruct((_B, 1, 128), f32),
    )(Y, Y, X, R_pred, tp3, R_gt, tg3, X_hat, X_hat, delta, delta)

    o = out[:, 0, :]
    NK = f32(_N * _K)
    L_align = o[:, 0] / NK
    L_deform = o[:, 1] / NK
    L_lap = o[:, 2] / f32(_N)
    L_disp = o[:, 3] / f32(_N)
    L_rmse = jnp.sqrt(o[:, 4] / f32(_N))
    tr = o[:, 5]
    trans_sq = o[:, 6]
    c = jnp.clip((tr - 1.0) / 2.0, -1.0 + 1e-07, 1.0 - 1e-07)
    L_rot = jnp.arccos(c)
    L_trans = jnp.sqrt(trans_sq)
    total = (L_rot + L_trans + L_rmse + L_align
             + 0.01 * L_disp + 0.1 * L_deform + 0.1 * L_lap)
    return total.mean()


# speedup vs baseline: 54.7805x; 1.1205x over previous
import jax
import jax.numpy as jnp
from jax import lax
from jax.experimental import pallas as pl
from jax.experimental.pallas import tpu as pltpu

_B, _N, _M, _K = 2, 2048, 1024, 5
_TILE = 1024
_T = _N // _TILE
_INF = 3.0e38
_SELF = 1.0e38


def _trace3(A, B3):
    P = lax.dot_general(A, B3, (((1,), (0,)), ((), ())))
    eye = (lax.broadcasted_iota(jnp.int32, (3, 3), 0)
           == lax.broadcasted_iota(jnp.int32, (3, 3), 1))
    return jnp.sum(jnp.where(eye, P, 0.0))


def _body(Yr_ref, Ya_ref, X_ref, Rp_ref, tp_ref, Rg_ref, tg_ref,
          Xhr_ref, Xha_ref, dlr_ref, dla_ref, out_ref):
    t = pl.program_id(1)
    f32 = jnp.float32
    i32 = jnp.int32

    yrow = Yr_ref[0]
    yall = Ya_ref[0]
    x = X_ref[0]
    Rp = Rp_ref[0]
    tp = tp_ref[0]
    Rg = Rg_ref[0]
    tg = tg_ref[0]
    xh_r = Xhr_ref[0]
    xh_a = Xha_ref[0]
    de_r = dlr_ref[0]
    de_a = dla_ref[0]

    mm = (((1,), (0,)), ((), ()))
    cT = (((0,), (0,)), ((), ()))

    yrig_r = lax.dot_general(Rp, yrow, mm) + tp
    yrig_a = lax.dot_general(Rp, yall, mm) + tp

    nr = jnp.sum(yrig_r * yrig_r, axis=0, keepdims=True)
    na = jnp.sum(yrig_a * yrig_a, axis=0, keepdims=True)
    ones_r = jnp.ones((1, _TILE), f32)
    ones_a = jnp.ones((1, _N), f32)
    U = jnp.concatenate([-2.0 * yrig_r, nr, ones_r], axis=0)
    V = jnp.concatenate([yrig_a, ones_a, na], axis=0)
    d = lax.dot_general(U, V, cT)

    row_id = t * _TILE + lax.broadcasted_iota(i32, (_TILE, _N), 0)
    col_id = lax.broadcasted_iota(i32, (_TILE, _N), 1)
    d = jnp.where(row_id == col_id, _SELF, d)

    m = jnp.min(d, axis=1, keepdims=True)
    for _ in range(_K - 1):
        m = jnp.min(jnp.where(d <= m, _INF, d), axis=1, keepdims=True)
    selmask = jnp.where(d <= m, 1.0, 0.0).astype(f32)

    D_a = xh_a - yrig_a
    D2_a = jnp.sum(D_a * D_a, axis=0, keepdims=True)
    F = jnp.concatenate([D_a, D2_a, de_a], axis=0)
    sel = lax.dot_general(selmask, F, (((1,), (1,)), ((), ())))
    S1 = sel[:, 0:3]
    S2 = sel[:, 3:4]
    Sd = sel[:, 4:7]

    D_r = xh_r - yrig_r
    deform_s = (jnp.sum(S2) - 2.0 * _trace3(D_r, S1)
                + f32(_K) * jnp.sum(D_r * D_r))
    disp_s = jnp.sum(de_r * de_r)
    lap_s = (disp_s - (2.0 / _K) * _trace3(de_r, Sd)
             + (1.0 / (_K * _K)) * jnp.sum(Sd * Sd))

    nxh = jnp.sum(xh_r * xh_r, axis=0, keepdims=True)
    nx = jnp.sum(x * x, axis=0, keepdims=True)
    ones_m = jnp.ones((1, _M), f32)
    UA = jnp.concatenate([-2.0 * xh_r, nxh, ones_r], axis=0)
    VA = jnp.concatenate([x, ones_m, nx], axis=0)
    dA = lax.dot_general(UA, VA, cT)
    mA = jnp.min(dA, axis=1, keepdims=True)
    align_s = jnp.sum(mA)
    for _ in range(_K - 1):
        mA = jnp.min(jnp.where(dA <= mA, _INF, dA), axis=1, keepdims=True)
        align_s = align_s + jnp.sum(mA)

    E = lax.dot_general(Rp - Rg, yrow, mm) + (tp - tg)
    rmse_s = jnp.sum(E * E)

    Rd = lax.dot_general(Rp, Rg, cT)
    eye = (lax.broadcasted_iota(i32, (3, 3), 0)
           == lax.broadcasted_iota(i32, (3, 3), 1))
    tr = jnp.sum(jnp.where(eye, Rd, 0.0))
    dtr = tp - tg
    trans_sq = jnp.sum(dtr * dtr)
    gate = jnp.where(t == 0, f32(1.0), f32(0.0))

    lane = lax.broadcasted_iota(i32, (1, 1, 128), 2)
    vals = (jnp.where(lane == 0, align_s, 0.0)
            + jnp.where(lane == 1, deform_s, 0.0)
            + jnp.where(lane == 2, lap_s, 0.0)
            + jnp.where(lane == 3, disp_s, 0.0)
            + jnp.where(lane == 4, rmse_s, 0.0)
            + jnp.where(lane == 5, gate * tr, 0.0)
            + jnp.where(lane == 6, gate * trans_sq, 0.0))

    @pl.when(t == 0)
    def _init():
        out_ref[...] = jnp.zeros_like(out_ref)

    out_ref[...] += vals


def kernel(Y, X, R_pred, t_pred, R_gt, t_gt, X_hat, delta):
    f32 = jnp.float32
    YT = jnp.swapaxes(Y, 1, 2)
    XT = jnp.swapaxes(X, 1, 2)
    XhT = jnp.swapaxes(X_hat, 1, 2)
    dlT = jnp.swapaxes(delta, 1, 2)
    tp3 = t_pred.reshape(_B, 3, 1).astype(f32)
    tg3 = t_gt.reshape(_B, 3, 1).astype(f32)

    rows = lambda b, t: (b, 0, t)
    full = lambda b, t: (b, 0, 0)

    out = pl.pallas_call(
        _body,
        grid=(_B, _T),
        in_specs=[
            pl.BlockSpec((1, 3, _TILE), rows),
            pl.BlockSpec((1, 3, _N), full),
            pl.BlockSpec((1, 3, _M), full),
            pl.BlockSpec((1, 3, 3), full),
            pl.BlockSpec((1, 3, 1), full),
            pl.BlockSpec((1, 3, 3), full),
            pl.BlockSpec((1, 3, 1), full),
            pl.BlockSpec((1, 3, _TILE), rows),
            pl.BlockSpec((1, 3, _N), full),
            pl.BlockSpec((1, 3, _TILE), rows),
            pl.BlockSpec((1, 3, _N), full),
        ],
        out_specs=pl.BlockSpec((1, 1, 128), full),
        out_shape=jax.ShapeDtypeStruct((_B, 1, 128), f32),
    )(YT, YT, XT, R_pred, tp3, R_gt, tg3, XhT, XhT, dlT, dlT)

    o = out[:, 0, :]
    NK = f32(_N * _K)
    L_align = o[:, 0] / NK
    L_deform = o[:, 1] / NK
    L_lap = o[:, 2] / f32(_N)
    L_disp = o[:, 3] / f32(_N)
    L_rmse = jnp.sqrt(o[:, 4] / f32(_N))
    tr = o[:, 5]
    trans_sq = o[:, 6]
    c = jnp.clip((tr - 1.0) / 2.0, -1.0 + 1e-07, 1.0 - 1e-07)
    L_rot = jnp.arccos(c)
    L_trans = jnp.sqrt(trans_sq)
    total = (L_rot + L_trans + L_rmse + L_align
             + 0.01 * L_disp + 0.1 * L_deform + 0.1 * L_lap)
    return total.mean()
